# baseline (speedup 1.0000x reference)
import functools
import math

import jax
import jax.numpy as jnp
from jax import lax
from jax.experimental import pallas as pl
from jax.experimental.pallas import tpu as pltpu
from jax.experimental.pallas import tpu_sc as plsc

D_MODEL = 64
SCALE = math.sqrt(D_MODEL)

NUM_CORES = 2
NUM_SUBCORES = 16
NUM_WORKERS = NUM_CORES * NUM_SUBCORES
LANES = 16


def _emb_kernel(b0_dim, b1_dim):
    blk = b0_dim // NUM_WORKERS
    assert blk % LANES == 0
    n_blk = blk // LANES
    mesh = plsc.VectorSubcoreMesh(core_axis_name="c", subcore_axis_name="s")

    @functools.partial(
        pl.kernel,
        mesh=mesh,
        out_type=jax.ShapeDtypeStruct((b1_dim, D_MODEL, b0_dim), jnp.float32),
        scratch_types=[
            pltpu.VMEM((b1_dim, blk), jnp.int32),
            pltpu.VMEM((blk, 2 * D_MODEL), jnp.float32),
            pltpu.VMEM((blk, 2 * D_MODEL), jnp.float32),
            pltpu.VMEM((D_MODEL, blk), jnp.float32),
            pltpu.VMEM((D_MODEL, blk), jnp.float32),
            pltpu.SemaphoreType.DMA,
            pltpu.SemaphoreType.DMA,
            pltpu.SemaphoreType.DMA,
            pltpu.SemaphoreType.DMA,
        ],
        compiler_params=pltpu.CompilerParams(
            use_tc_tiling_on_sc=True, needs_layout_passes=False
        ),
    )
    def k(xt_hbm, table_hbm, out_hbm, idx_v, g0, g1, t0, t1,
          gs0, gs1, ss0, ss1):
        cid = lax.axis_index("c")
        sid = lax.axis_index("s")
        wid = sid * NUM_CORES + cid
        col0 = wid * blk

        pltpu.sync_copy(xt_hbm.at[:, pl.ds(col0, blk)], idx_v)

        def gather(b1, g, sem):
            b1c = jnp.minimum(b1, b1_dim - 1)
            return pltpu.make_async_copy(
                table_hbm.at[idx_v.at[b1c]], g, sem
            )

        def store(b1, t, sem):
            return pltpu.make_async_copy(
                t, out_hbm.at[b1, :, pl.ds(col0, blk)], sem
            )

        iota = lax.iota(jnp.int32, LANES)
        rowsel = [iota + b * LANES for b in range(n_blk)]
        zeros = jnp.zeros((LANES,), jnp.int32)

        def transpose_scale(g, t):
            def d_body(d, carry):
                colsel = zeros + d
                vals = [
                    plsc.load_gather(g, [rowsel[b], colsel])
                    for b in range(n_blk)
                ]
                for b in range(n_blk):
                    t[d, pl.ds(b * LANES, LANES)] = vals[b] * SCALE
                return carry

            lax.fori_loop(0, D_MODEL, d_body, 0)

        gather(0, g0, gs0).start()
        gather(1, g1, gs1).start()

        def body(j, carry):
            b1 = j * 2
            gather(b1, g0, gs0).wait()
            transpose_scale(g0, t0)
            store(b1, t0, ss0).start()
            gather(b1 + 2, g0, gs0).start()
            gather(b1 + 1, g1, gs1).wait()
            transpose_scale(g1, t1)
            store(b1 + 1, t1, ss1).start()
            gather(b1 + 3, g1, gs1).start()
            store(b1, t0, ss0).wait()
            store(b1 + 1, t1, ss1).wait()
            return carry

        lax.fori_loop(0, b1_dim // 2, body, 0)

        gather(b1_dim - 1, g0, gs0).wait()
        gather(b1_dim - 1, g1, gs1).wait()

    return k


def kernel(x, table):
    b0, b1 = x.shape
    xt = jnp.swapaxes(x, 0, 1).astype(jnp.int32)
    tpad = jnp.pad(table, ((0, 0), (0, D_MODEL)))
    out = _emb_kernel(b0, b1)(xt, tpad)
    return jnp.transpose(out, (2, 0, 1))


# speedup vs baseline: 1.2578x; 1.2578x over previous
import functools
import math

import jax
import jax.numpy as jnp
from jax import lax
from jax.experimental import pallas as pl
from jax.experimental.pallas import tpu as pltpu
from jax.experimental.pallas import tpu_sc as plsc

D_MODEL = 64
SCALE = math.sqrt(D_MODEL)

NUM_CORES = 2
NUM_SUBCORES = 16
NUM_WORKERS = NUM_CORES * NUM_SUBCORES
LANES = 16

CHUNK = 800


def _emb_kernel(n_rows):
    b_per_w = n_rows // NUM_WORKERS
    n_chunks = b_per_w // CHUNK
    assert n_chunks * CHUNK == b_per_w and n_chunks % 2 == 0
    assert CHUNK % 200 == 0
    n_b0 = CHUNK // 200
    mesh = plsc.VectorSubcoreMesh(core_axis_name="c", subcore_axis_name="s")

    @functools.partial(
        pl.kernel,
        mesh=mesh,
        out_type=jax.ShapeDtypeStruct((n_rows // 200, 200, D_MODEL),
                                      jnp.float32),
        scratch_types=[
            pltpu.VMEM((b_per_w,), jnp.int32),
            pltpu.VMEM((CHUNK, D_MODEL), jnp.float32),
            pltpu.VMEM((CHUNK, D_MODEL), jnp.float32),
            pltpu.SemaphoreType.DMA,
            pltpu.SemaphoreType.DMA,
            pltpu.SemaphoreType.DMA,
            pltpu.SemaphoreType.DMA,
        ],
        compiler_params=pltpu.CompilerParams(use_tc_tiling_on_sc=False),
    )
    def k(x_hbm, table_hbm, out3_hbm, idx_v, rows0, rows1, g0, g1, s0, s1):
        cid = lax.axis_index("c")
        sid = lax.axis_index("s")
        wid = sid * NUM_CORES + cid
        base = wid * b_per_w

        pltpu.sync_copy(x_hbm.at[pl.ds(base, b_per_w)], idx_v)

        def gather(i, rows, sem):
            ic = jnp.minimum(i, n_chunks - 1)
            return pltpu.make_async_copy(
                table_hbm.at[idx_v.at[pl.ds(ic * CHUNK, CHUNK)]], rows, sem
            )

        class _StoreGroup:
            def __init__(self, i, rows, sem):
                b0_0 = wid * (b_per_w // 200) + i * n_b0
                self.copies = [
                    pltpu.make_async_copy(
                        rows.at[pl.ds(r * 200, 200)],
                        out3_hbm.at[b0_0 + r],
                        sem,
                    )
                    for r in range(n_b0)
                ]

            def start(self):
                for c in self.copies:
                    c.start()

            def wait(self):
                for c in self.copies:
                    c.wait()

        store = _StoreGroup

        def scale(rows):
            def scale_row(r, carry):
                for c4 in range(D_MODEL // LANES):
                    sl = pl.ds(c4 * LANES, LANES)
                    rows[r, sl] = rows[r, sl] * SCALE
                return carry

            lax.fori_loop(0, CHUNK, scale_row, 0, unroll=4)

        gather(0, rows0, g0).start()
        gather(1, rows1, g1).start()

        def body(j, carry):
            i = j * 2
            gather(i, rows0, g0).wait()
            scale(rows0)
            store(i, rows0, s0).start()
            gather(i + 1, rows1, g1).wait()
            scale(rows1)
            store(i + 1, rows1, s1).start()
            store(i, rows0, s0).wait()
            gather(i + 2, rows0, g0).start()
            store(i + 1, rows1, s1).wait()
            gather(i + 3, rows1, g1).start()
            return carry

        lax.fori_loop(0, n_chunks // 2, body, 0)

        gather(n_chunks - 1, rows0, g0).wait()
        gather(n_chunks - 1, rows1, g1).wait()

    return k


def kernel(x, table):
    b0, b1 = x.shape
    n_rows = b0 * b1
    out = _emb_kernel(n_rows)(x.reshape(n_rows).astype(jnp.int32), table)
    return out.reshape(b0, b1, D_MODEL)
